# baseline (speedup 1.0000x reference)
import functools

import jax
import jax.numpy as jnp
from jax import lax
from jax.experimental import pallas as pl
from jax.experimental.pallas import tpu as pltpu
from jax.experimental.pallas import tpu_sc as plsc

N = 10000
E = 320000
D = 128
G = 16
K = 5

NC = 2
NS = 16
NW = NC * NS
LANES = 16

NP = 10240
CHUNK = 128
CPW = 80
HCPW = CPW // 2
EPW = CPW * CHUNK
EP = NW * EPW
RPT = NP // NS
DPT = NP // NS

_mesh = plsc.VectorSubcoreMesh(core_axis_name="c", subcore_axis_name="s")
_sc_params = pltpu.CompilerParams(needs_layout_passes=False)


def _splat(v):
    return jnp.full((LANES,), v, jnp.int32)


def _fast_rsqrt(x):
    s = 0.5 * (x + 1.0)
    for _ in range(16):
        s = 0.5 * (s + x / s)
    return 1.0 / s


@functools.partial(
    pl.kernel,
    out_type=jax.ShapeDtypeStruct((NW, CPW, CHUNK), jnp.float32),
    mesh=_mesh,
    compiler_params=_sc_params,
    scratch_types=[
        pltpu.VMEM((CPW, CHUNK), jnp.int32),
        pltpu.VMEM((CPW, CHUNK), jnp.float32),
        pltpu.VMEM((CPW, CHUNK), jnp.int32),
        pltpu.VMEM((CPW, CHUNK), jnp.float32),
        pltpu.VMEM((CPW, CHUNK), jnp.int32),
        pltpu.VMEM((CPW, CHUNK), jnp.int32),
        pltpu.VMEM((CPW, CHUNK), jnp.float32),
        pltpu.VMEM((NP,), jnp.float32),
        pltpu.VMEM((CPW, CHUNK), jnp.float32),
        pltpu.VMEM((DPT,), jnp.float32),
        pltpu.VMEM((DPT,), jnp.float32),
        pltpu.VMEM_SHARED((NP,), jnp.float32),
        pltpu.VMEM_SHARED((NP,), jnp.float32),
    ],
)
def _prep(src_h, dst_h, ew_h, wn_h,
          src_a, ew_a, src_b, ew_b, src_c, dst_c, ew_c,
          dis_v, wn_v, deg_s, dis_sv, deg_sh, dis_sh):
    c = lax.axis_index("c")
    s = lax.axis_index("s")
    g = c * NS + s

    zeros16 = jnp.zeros((LANES,), jnp.float32)

    def z_body(i, _):
        deg_s[pl.ds(i * LANES, LANES)] = zeros16
        return 0
    lax.fori_loop(0, DPT // LANES, z_body, 0)
    pltpu.sync_copy(deg_s, deg_sh.at[pl.ds(s * DPT, DPT)])
    plsc.subcore_barrier()

    pltpu.sync_copy(src_h.at[s], src_a)
    pltpu.sync_copy(ew_h.at[s], ew_a)
    pltpu.sync_copy(src_h.at[s + NS], src_b)
    pltpu.sync_copy(ew_h.at[s + NS], ew_b)

    def deg_body(i, _):
        pltpu.sync_copy(ew_a.at[i], deg_sh.at[src_a.at[i]], add=True)
        pltpu.sync_copy(ew_b.at[i], deg_sh.at[src_b.at[i]], add=True)
        return 0
    lax.fori_loop(0, CPW, deg_body, 0)
    plsc.subcore_barrier()

    pltpu.sync_copy(deg_sh.at[pl.ds(s * DPT, DPT)], deg_s)

    def dis_body(i, _):
        d16 = deg_s[pl.ds(i * LANES, LANES)]
        r = _fast_rsqrt(d16)
        dis_sv[pl.ds(i * LANES, LANES)] = jnp.where(d16 > 0.0, r, 0.0)
        return 0
    lax.fori_loop(0, DPT // LANES, dis_body, 0)
    pltpu.sync_copy(dis_sv, dis_sh.at[pl.ds(s * DPT, DPT)])
    plsc.subcore_barrier()

    pltpu.sync_copy(dis_sh, dis_v)
    pltpu.sync_copy(src_h.at[g], src_c)
    pltpu.sync_copy(dst_h.at[g], dst_c)
    pltpu.sync_copy(ew_h.at[g], ew_c)

    def wn_body(i, _):
        def grp(k, _):
            sl = pl.ds(k * LANES, LANES)
            s16 = src_c[i, sl]
            d16 = dst_c[i, sl]
            w16 = ew_c[i, sl]
            a = plsc.load_gather(dis_v, [s16])
            b = plsc.load_gather(dis_v, [d16])
            wn_v[i, sl] = -(a * w16 * b)
            return 0
        lax.fori_loop(0, CHUNK // LANES, grp, 0)
        return 0
    lax.fori_loop(0, CPW, wn_body, 0)
    pltpu.sync_copy(wn_v, wn_h.at[g])


@functools.partial(
    pl.kernel,
    out_type=jax.ShapeDtypeStruct((NC, NP, D), jnp.float32),
    mesh=_mesh,
    compiler_params=_sc_params,
    scratch_types=[
        pltpu.VMEM((HCPW, CHUNK), jnp.int32),
        pltpu.VMEM((HCPW, CHUNK), jnp.int32),
        pltpu.VMEM((HCPW, CHUNK), jnp.float32),
        pltpu.VMEM((CHUNK, D), jnp.float32),
        pltpu.VMEM((CHUNK, D), jnp.float32),
        pltpu.VMEM_SHARED((NP, D), jnp.float32),
        pltpu.SemaphoreType.DMA,
        pltpu.SemaphoreType.DMA,
        pltpu.SemaphoreType.DMA,
        pltpu.SemaphoreType.DMA,
    ],
)
def _prop(t_h, src_h, dst_h, wn_h, out_h, src_v, dst_v, wn_v, rows0, rows1,
          acc_sh, sg0, sg1, ss0, ss1):
    c = lax.axis_index("c")
    s = lax.axis_index("s")
    g = s * NC + c
    bufs = (rows0, rows1)
    gsems = (sg0, sg1)
    ssems = (ss0, ss1)

    zeros16 = jnp.zeros((LANES,), jnp.float32)

    def zrow(i, _):
        for j in range(D // LANES):
            rows0[i, pl.ds(j * LANES, LANES)] = zeros16
        return 0
    lax.fori_loop(0, CHUNK, zrow, 0)
    for b in range(RPT // CHUNK):
        pltpu.sync_copy(rows0, acc_sh.at[pl.ds(s * RPT + b * CHUNK, CHUNK)])
    plsc.subcore_barrier()

    for h in (0, 1):
        pltpu.sync_copy(src_h.at[g, pl.ds(h * HCPW, HCPW)], src_v)
        pltpu.sync_copy(dst_h.at[g, pl.ds(h * HCPW, HCPW)], dst_v)
        pltpu.sync_copy(wn_h.at[g, pl.ds(h * HCPW, HCPW)], wn_v)

        def chunk_body(i, _):
            pltpu.async_copy(t_h.at[src_v.at[i]], rows0, sg0).wait()

            def edge(e, _):
                wb = plsc.load_gather(wn_v, [_splat(i), _splat(e)])
                for j in range(D // LANES):
                    sl = pl.ds(j * LANES, LANES)
                    rows0[e, sl] = rows0[e, sl] * wb
                return 0
            lax.fori_loop(0, CHUNK, edge, 0)
            pltpu.sync_copy(rows0, acc_sh.at[dst_v.at[i]], add=True)
            return 0
        lax.fori_loop(0, HCPW, chunk_body, 0)
    plsc.subcore_barrier()

    for b in range(RPT // CHUNK):
        r0 = s * RPT + b * CHUNK
        pltpu.sync_copy(acc_sh.at[pl.ds(r0, CHUNK)],
                        out_h.at[c, pl.ds(r0, CHUNK), :])


_BLK = 512
_NBLK = NP // _BLK


def _combine1_body(p0_ref, p1_ref, o_ref):
    o_ref[...] = p0_ref[...] + p1_ref[...]


def _combine2_body(p0_ref, p1_ref, prev_ref, o_ref):
    o_ref[...] = 2.0 * (p0_ref[...] + p1_ref[...]) - prev_ref[...]


def _combine1(p0, p1):
    return pl.pallas_call(
        _combine1_body,
        out_shape=jax.ShapeDtypeStruct((NP, D), jnp.float32),
        grid=(_NBLK,),
        in_specs=[pl.BlockSpec((_BLK, D), lambda i: (i, 0))] * 2,
        out_specs=pl.BlockSpec((_BLK, D), lambda i: (i, 0)),
    )(p0, p1)


def _combine2(p0, p1, prev):
    return pl.pallas_call(
        _combine2_body,
        out_shape=jax.ShapeDtypeStruct((NP, D), jnp.float32),
        grid=(_NBLK,),
        in_specs=[pl.BlockSpec((_BLK, D), lambda i: (i, 0))] * 3,
        out_specs=pl.BlockSpec((_BLK, D), lambda i: (i, 0)),
    )(p0, p1, prev)


def _matmul5_body(relu, t0, t1, t2, t3, t4, w_ref, b_ref, o_ref):
    acc = jnp.dot(t0[...], w_ref[0], preferred_element_type=jnp.float32)
    for k, t in enumerate((t1, t2, t3, t4)):
        acc += jnp.dot(t[...], w_ref[k + 1],
                       preferred_element_type=jnp.float32)
    acc = acc + b_ref[...]
    if relu:
        acc = jnp.maximum(acc, 0.0)
    o_ref[...] = acc


def _matmul5(txs, W, b2d, relu):
    return pl.pallas_call(
        functools.partial(_matmul5_body, relu),
        out_shape=jax.ShapeDtypeStruct((NP, D), jnp.float32),
        grid=(_NBLK,),
        in_specs=[pl.BlockSpec((_BLK, D), lambda i: (i, 0))] * 5
        + [pl.BlockSpec((K, D, D), lambda i: (0, 0, 0)),
           pl.BlockSpec((1, D), lambda i: (0, 0))],
        out_specs=pl.BlockSpec((_BLK, D), lambda i: (i, 0)),
    )(*txs, W, b2d)


def _pool_head_body(h_ref, b3_ref, w1_ref, b1_ref, w2_ref, b2_ref, o_ref,
                    pooled, cnt):
    i = pl.program_id(0)

    @pl.when(i == 0)
    def _():
        pooled[...] = jnp.zeros_like(pooled)
        cnt[...] = jnp.zeros_like(cnt)

    bvec = b3_ref[0, 0, :]
    classes = lax.broadcasted_iota(jnp.int32, (_BLK, G), 1)
    onehot = (bvec[:, None] == classes).astype(jnp.float32)
    pooled[...] += jnp.dot(onehot.T, h_ref[...],
                           preferred_element_type=jnp.float32)
    cnt[...] += jnp.broadcast_to(
        jnp.sum(onehot, axis=0)[:, None], (G, D))

    @pl.when(i == _NBLK - 1)
    def _():
        p = pooled[...] / jnp.maximum(cnt[...], 1.0)
        p = jnp.maximum(p, 0.0)
        z = jnp.dot(p, w1_ref[...], preferred_element_type=jnp.float32)
        z = z + b1_ref[...]
        z = jnp.dot(z, w2_ref[...], preferred_element_type=jnp.float32)
        z = z + b2_ref[...]
        o_ref[...] = 1.0 / (1.0 + jnp.exp(-z))


def _pool_head(h, batch3, lin1_W, b1_2d, lin2_p, b2_2d):
    return pl.pallas_call(
        _pool_head_body,
        out_shape=jax.ShapeDtypeStruct((G, D), jnp.float32),
        grid=(_NBLK,),
        in_specs=[
            pl.BlockSpec((_BLK, D), lambda i: (i, 0)),
            pl.BlockSpec((1, 1, _BLK), lambda i: (i, 0, 0)),
            pl.BlockSpec((D, D), lambda i: (0, 0)),
            pl.BlockSpec((1, D), lambda i: (0, 0)),
            pl.BlockSpec((D, D), lambda i: (0, 0)),
            pl.BlockSpec((1, D), lambda i: (0, 0)),
        ],
        out_specs=pl.BlockSpec((G, D), lambda i: (0, 0)),
        scratch_shapes=[
            pltpu.VMEM((G, D), jnp.float32),
            pltpu.VMEM((G, D), jnp.float32),
        ],
    )(h, batch3, lin1_W, b1_2d, lin2_p, b2_2d)


def _cheb_layer(h, src3, dst3, wn3, W, b2d, relu):
    tx0 = h
    p = _prop(tx0, src3, dst3, wn3)
    tx1 = _combine1(p[0], p[1])
    p = _prop(tx1, src3, dst3, wn3)
    tx2 = _combine2(p[0], p[1], tx0)
    p = _prop(tx2, src3, dst3, wn3)
    tx3 = _combine2(p[0], p[1], tx1)
    p = _prop(tx3, src3, dst3, wn3)
    tx4 = _combine2(p[0], p[1], tx2)
    return _matmul5((tx0, tx1, tx2, tx3, tx4), W, b2d, relu)


def kernel(x, edge_index, edge_weight, batch,
           conv1_W, conv1_b, conv2_W, conv2_b, conv3_W, conv3_b,
           lin1_W, lin1_b, lin2_W, lin2_b):
    src = edge_index[0]
    dst = edge_index[1]
    pad_e = EP - E
    src3 = jnp.pad(src, (0, pad_e)).reshape(NW, CPW, CHUNK)
    dst3 = jnp.pad(dst, (0, pad_e)).reshape(NW, CPW, CHUNK)
    ew3 = jnp.pad(edge_weight, (0, pad_e)).reshape(NW, CPW, CHUNK)
    x_p = jnp.pad(x, ((0, NP - N), (0, 0)))
    batch3 = jnp.pad(batch, (0, NP - N), constant_values=G).reshape(
        _NBLK, 1, _BLK)
    b1c = conv1_b.reshape(1, D)
    b2c = conv2_b.reshape(1, D)
    b3c = conv3_b.reshape(1, D)
    l1b = lin1_b.reshape(1, D)
    l2w = jnp.pad(lin2_W, ((0, 0), (0, D - 1)))
    l2b = jnp.pad(lin2_b, (0, D - 1)).reshape(1, D)

    wn3 = _prep(src3, dst3, ew3)

    h = _cheb_layer(x_p, src3, dst3, wn3, conv1_W, b1c, relu=True)
    h = _cheb_layer(h, src3, dst3, wn3, conv2_W, b2c, relu=False)
    h = _cheb_layer(h, src3, dst3, wn3, conv3_W, b3c, relu=True)

    out = _pool_head(h, batch3, lin1_W, l1b, l2w, l2b)
    return out[:, :1]


# speedup vs baseline: 1.0021x; 1.0021x over previous
import functools

import jax
import jax.numpy as jnp
from jax import lax
from jax.experimental import pallas as pl
from jax.experimental.pallas import tpu as pltpu
from jax.experimental.pallas import tpu_sc as plsc

N = 10000
E = 320000
D = 128
G = 16
K = 5

NC = 2
NS = 16
NW = NC * NS
LANES = 16

NP = 10240
CHUNK = 128
CPW = 80
HCPW = CPW // 2
EPW = CPW * CHUNK
EP = NW * EPW
RPT = NP // NS
DPT = NP // NS

_mesh = plsc.VectorSubcoreMesh(core_axis_name="c", subcore_axis_name="s")
_sc_params = pltpu.CompilerParams(needs_layout_passes=False)


def _splat(v):
    return jnp.full((LANES,), v, jnp.int32)


def _fast_rsqrt(x):
    s = 0.5 * (x + 1.0)
    for _ in range(16):
        s = 0.5 * (s + x / s)
    return 1.0 / s


@functools.partial(
    pl.kernel,
    out_type=jax.ShapeDtypeStruct((NW, CPW, CHUNK), jnp.float32),
    mesh=_mesh,
    compiler_params=_sc_params,
    scratch_types=[
        pltpu.VMEM((CPW, CHUNK), jnp.int32),
        pltpu.VMEM((CPW, CHUNK), jnp.float32),
        pltpu.VMEM((CPW, CHUNK), jnp.int32),
        pltpu.VMEM((CPW, CHUNK), jnp.float32),
        pltpu.VMEM((CPW, CHUNK), jnp.int32),
        pltpu.VMEM((CPW, CHUNK), jnp.int32),
        pltpu.VMEM((CPW, CHUNK), jnp.float32),
        pltpu.VMEM((NP,), jnp.float32),
        pltpu.VMEM((CPW, CHUNK), jnp.float32),
        pltpu.VMEM((DPT,), jnp.float32),
        pltpu.VMEM((DPT,), jnp.float32),
        pltpu.VMEM_SHARED((NP,), jnp.float32),
        pltpu.VMEM_SHARED((NP,), jnp.float32),
    ],
)
def _prep(src_h, dst_h, ew_h, wn_h,
          src_a, ew_a, src_b, ew_b, src_c, dst_c, ew_c,
          dis_v, wn_v, deg_s, dis_sv, deg_sh, dis_sh):
    c = lax.axis_index("c")
    s = lax.axis_index("s")
    g = c * NS + s

    zeros16 = jnp.zeros((LANES,), jnp.float32)

    def z_body(i, _):
        deg_s[pl.ds(i * LANES, LANES)] = zeros16
        return 0
    lax.fori_loop(0, DPT // LANES, z_body, 0)
    pltpu.sync_copy(deg_s, deg_sh.at[pl.ds(s * DPT, DPT)])
    plsc.subcore_barrier()

    pltpu.sync_copy(src_h.at[s], src_a)
    pltpu.sync_copy(ew_h.at[s], ew_a)
    pltpu.sync_copy(src_h.at[s + NS], src_b)
    pltpu.sync_copy(ew_h.at[s + NS], ew_b)

    def deg_body(i, _):
        pltpu.sync_copy(ew_a.at[i], deg_sh.at[src_a.at[i]], add=True)
        pltpu.sync_copy(ew_b.at[i], deg_sh.at[src_b.at[i]], add=True)
        return 0
    lax.fori_loop(0, CPW, deg_body, 0)
    plsc.subcore_barrier()

    pltpu.sync_copy(deg_sh.at[pl.ds(s * DPT, DPT)], deg_s)

    def dis_body(i, _):
        d16 = deg_s[pl.ds(i * LANES, LANES)]
        r = _fast_rsqrt(d16)
        dis_sv[pl.ds(i * LANES, LANES)] = jnp.where(d16 > 0.0, r, 0.0)
        return 0
    lax.fori_loop(0, DPT // LANES, dis_body, 0)
    pltpu.sync_copy(dis_sv, dis_sh.at[pl.ds(s * DPT, DPT)])
    plsc.subcore_barrier()

    pltpu.sync_copy(dis_sh, dis_v)
    pltpu.sync_copy(src_h.at[g], src_c)
    pltpu.sync_copy(dst_h.at[g], dst_c)
    pltpu.sync_copy(ew_h.at[g], ew_c)

    def wn_body(i, _):
        def grp(k, _):
            sl = pl.ds(k * LANES, LANES)
            s16 = src_c[i, sl]
            d16 = dst_c[i, sl]
            w16 = ew_c[i, sl]
            a = plsc.load_gather(dis_v, [s16])
            b = plsc.load_gather(dis_v, [d16])
            wn_v[i, sl] = -(a * w16 * b)
            return 0
        lax.fori_loop(0, CHUNK // LANES, grp, 0)
        return 0
    lax.fori_loop(0, CPW, wn_body, 0)
    pltpu.sync_copy(wn_v, wn_h.at[g])


@functools.partial(
    pl.kernel,
    out_type=jax.ShapeDtypeStruct((NC, NP, D), jnp.float32),
    mesh=_mesh,
    compiler_params=_sc_params,
    scratch_types=[
        pltpu.VMEM((CPW, CHUNK), jnp.int32),
        pltpu.VMEM((CPW, CHUNK), jnp.int32),
        pltpu.VMEM((CPW, CHUNK), jnp.float32),
        pltpu.VMEM((CHUNK, D), jnp.float32),
        pltpu.VMEM_SHARED((NP, D), jnp.float32),
        pltpu.SemaphoreType.DMA,
    ],
)
def _prop(t_h, src_h, dst_h, wn_h, out_h, src_v, dst_v, wn_v, rows0,
          acc_sh, sg0):
    c = lax.axis_index("c")
    s = lax.axis_index("s")
    g = s * NC + c

    zeros16 = jnp.zeros((LANES,), jnp.float32)

    def zrow(i, _):
        for j in range(D // LANES):
            rows0[i, pl.ds(j * LANES, LANES)] = zeros16
        return 0
    lax.fori_loop(0, CHUNK, zrow, 0)
    for b in range(RPT // CHUNK):
        pltpu.sync_copy(rows0, acc_sh.at[pl.ds(s * RPT + b * CHUNK, CHUNK)])
    plsc.subcore_barrier()

    pltpu.sync_copy(src_h.at[g], src_v)
    pltpu.sync_copy(dst_h.at[g], dst_v)
    pltpu.sync_copy(wn_h.at[g], wn_v)

    def chunk_body(i, _):
        pltpu.async_copy(t_h.at[src_v.at[i]], rows0, sg0).wait()

        def edge(e, _):
            wb = plsc.load_gather(wn_v, [_splat(i), _splat(e)])
            for j in range(D // LANES):
                sl = pl.ds(j * LANES, LANES)
                rows0[e, sl] = rows0[e, sl] * wb
            return 0
        lax.fori_loop(0, CHUNK, edge, 0)
        pltpu.sync_copy(rows0, acc_sh.at[dst_v.at[i]], add=True)
        return 0
    lax.fori_loop(0, CPW, chunk_body, 0)
    plsc.subcore_barrier()

    for b in range(RPT // CHUNK):
        r0 = s * RPT + b * CHUNK
        pltpu.sync_copy(acc_sh.at[pl.ds(r0, CHUNK)],
                        out_h.at[c, pl.ds(r0, CHUNK), :])


_BLK = 512
_NBLK = NP // _BLK


def _combine1_body(p0_ref, p1_ref, o_ref):
    o_ref[...] = p0_ref[...] + p1_ref[...]


def _combine2_body(p0_ref, p1_ref, prev_ref, o_ref):
    o_ref[...] = 2.0 * (p0_ref[...] + p1_ref[...]) - prev_ref[...]


def _combine1(p0, p1):
    return pl.pallas_call(
        _combine1_body,
        out_shape=jax.ShapeDtypeStruct((NP, D), jnp.float32),
        grid=(_NBLK,),
        in_specs=[pl.BlockSpec((_BLK, D), lambda i: (i, 0))] * 2,
        out_specs=pl.BlockSpec((_BLK, D), lambda i: (i, 0)),
    )(p0, p1)


def _combine2(p0, p1, prev):
    return pl.pallas_call(
        _combine2_body,
        out_shape=jax.ShapeDtypeStruct((NP, D), jnp.float32),
        grid=(_NBLK,),
        in_specs=[pl.BlockSpec((_BLK, D), lambda i: (i, 0))] * 3,
        out_specs=pl.BlockSpec((_BLK, D), lambda i: (i, 0)),
    )(p0, p1, prev)


def _matmul5_body(relu, t0, t1, t2, t3, t4, w_ref, b_ref, o_ref):
    acc = jnp.dot(t0[...], w_ref[0], preferred_element_type=jnp.float32)
    for k, t in enumerate((t1, t2, t3, t4)):
        acc += jnp.dot(t[...], w_ref[k + 1],
                       preferred_element_type=jnp.float32)
    acc = acc + b_ref[...]
    if relu:
        acc = jnp.maximum(acc, 0.0)
    o_ref[...] = acc


def _matmul5(txs, W, b2d, relu):
    return pl.pallas_call(
        functools.partial(_matmul5_body, relu),
        out_shape=jax.ShapeDtypeStruct((NP, D), jnp.float32),
        grid=(_NBLK,),
        in_specs=[pl.BlockSpec((_BLK, D), lambda i: (i, 0))] * 5
        + [pl.BlockSpec((K, D, D), lambda i: (0, 0, 0)),
           pl.BlockSpec((1, D), lambda i: (0, 0))],
        out_specs=pl.BlockSpec((_BLK, D), lambda i: (i, 0)),
    )(*txs, W, b2d)


def _pool_head_body(h_ref, b3_ref, w1_ref, b1_ref, w2_ref, b2_ref, o_ref,
                    pooled, cnt):
    i = pl.program_id(0)

    @pl.when(i == 0)
    def _():
        pooled[...] = jnp.zeros_like(pooled)
        cnt[...] = jnp.zeros_like(cnt)

    bvec = b3_ref[0, 0, :]
    classes = lax.broadcasted_iota(jnp.int32, (_BLK, G), 1)
    onehot = (bvec[:, None] == classes).astype(jnp.float32)
    pooled[...] += jnp.dot(onehot.T, h_ref[...],
                           preferred_element_type=jnp.float32)
    cnt[...] += jnp.broadcast_to(
        jnp.sum(onehot, axis=0)[:, None], (G, D))

    @pl.when(i == _NBLK - 1)
    def _():
        p = pooled[...] / jnp.maximum(cnt[...], 1.0)
        p = jnp.maximum(p, 0.0)
        z = jnp.dot(p, w1_ref[...], preferred_element_type=jnp.float32)
        z = z + b1_ref[...]
        z = jnp.dot(z, w2_ref[...], preferred_element_type=jnp.float32)
        z = z + b2_ref[...]
        o_ref[...] = 1.0 / (1.0 + jnp.exp(-z))


def _pool_head(h, batch3, lin1_W, b1_2d, lin2_p, b2_2d):
    return pl.pallas_call(
        _pool_head_body,
        out_shape=jax.ShapeDtypeStruct((G, D), jnp.float32),
        grid=(_NBLK,),
        in_specs=[
            pl.BlockSpec((_BLK, D), lambda i: (i, 0)),
            pl.BlockSpec((1, 1, _BLK), lambda i: (i, 0, 0)),
            pl.BlockSpec((D, D), lambda i: (0, 0)),
            pl.BlockSpec((1, D), lambda i: (0, 0)),
            pl.BlockSpec((D, D), lambda i: (0, 0)),
            pl.BlockSpec((1, D), lambda i: (0, 0)),
        ],
        out_specs=pl.BlockSpec((G, D), lambda i: (0, 0)),
        scratch_shapes=[
            pltpu.VMEM((G, D), jnp.float32),
            pltpu.VMEM((G, D), jnp.float32),
        ],
    )(h, batch3, lin1_W, b1_2d, lin2_p, b2_2d)


def _cheb_layer(h, src3, dst3, wn3, W, b2d, relu):
    tx0 = h
    p = _prop(tx0, src3, dst3, wn3)
    tx1 = _combine1(p[0], p[1])
    p = _prop(tx1, src3, dst3, wn3)
    tx2 = _combine2(p[0], p[1], tx0)
    p = _prop(tx2, src3, dst3, wn3)
    tx3 = _combine2(p[0], p[1], tx1)
    p = _prop(tx3, src3, dst3, wn3)
    tx4 = _combine2(p[0], p[1], tx2)
    return _matmul5((tx0, tx1, tx2, tx3, tx4), W, b2d, relu)


def kernel(x, edge_index, edge_weight, batch,
           conv1_W, conv1_b, conv2_W, conv2_b, conv3_W, conv3_b,
           lin1_W, lin1_b, lin2_W, lin2_b):
    src = edge_index[0]
    dst = edge_index[1]
    pad_e = EP - E
    src3 = jnp.pad(src, (0, pad_e)).reshape(NW, CPW, CHUNK)
    dst3 = jnp.pad(dst, (0, pad_e)).reshape(NW, CPW, CHUNK)
    ew3 = jnp.pad(edge_weight, (0, pad_e)).reshape(NW, CPW, CHUNK)
    x_p = jnp.pad(x, ((0, NP - N), (0, 0)))
    batch3 = jnp.pad(batch, (0, NP - N), constant_values=G).reshape(
        _NBLK, 1, _BLK)
    b1c = conv1_b.reshape(1, D)
    b2c = conv2_b.reshape(1, D)
    b3c = conv3_b.reshape(1, D)
    l1b = lin1_b.reshape(1, D)
    l2w = jnp.pad(lin2_W, ((0, 0), (0, D - 1)))
    l2b = jnp.pad(lin2_b, (0, D - 1)).reshape(1, D)

    wn3 = _prep(src3, dst3, ew3)

    h = _cheb_layer(x_p, src3, dst3, wn3, conv1_W, b1c, relu=True)
    h = _cheb_layer(h, src3, dst3, wn3, conv2_W, b2c, relu=False)
    h = _cheb_layer(h, src3, dst3, wn3, conv3_W, b3c, relu=True)

    out = _pool_head(h, batch3, lin1_W, l1b, l2w, l2b)
    return out[:, :1]


# speedup vs baseline: 1.4025x; 1.3995x over previous
import functools

import jax
import jax.numpy as jnp
from jax import lax
from jax.experimental import pallas as pl
from jax.experimental.pallas import tpu as pltpu
from jax.experimental.pallas import tpu_sc as plsc

N = 10000
E = 320000
D = 128
G = 16
K = 5

NC = 2
NS = 16
NW = NC * NS
LANES = 16

NP = 10240
CHUNK = 128
CPW = 79
EPW = CPW * CHUNK
EP = NW * EPW
RPT = NP // NS
DPT = NP // NS

_mesh = plsc.VectorSubcoreMesh(core_axis_name="c", subcore_axis_name="s")
_sc_params = pltpu.CompilerParams(needs_layout_passes=False)


def _splat(v):
    return jnp.full((LANES,), v, jnp.int32)


def _fast_rsqrt(x):
    s = 0.5 * (x + 1.0)
    for _ in range(16):
        s = 0.5 * (s + x / s)
    return 1.0 / s


@functools.partial(
    pl.kernel,
    out_type=jax.ShapeDtypeStruct((NW, CPW, CHUNK), jnp.float32),
    mesh=_mesh,
    compiler_params=_sc_params,
    scratch_types=[
        pltpu.VMEM((CPW, CHUNK), jnp.int32),
        pltpu.VMEM((CPW, CHUNK), jnp.float32),
        pltpu.VMEM((CPW, CHUNK), jnp.int32),
        pltpu.VMEM((CPW, CHUNK), jnp.float32),
        pltpu.VMEM((CPW, CHUNK), jnp.int32),
        pltpu.VMEM((CPW, CHUNK), jnp.int32),
        pltpu.VMEM((CPW, CHUNK), jnp.float32),
        pltpu.VMEM((NP,), jnp.float32),
        pltpu.VMEM((CPW, CHUNK), jnp.float32),
        pltpu.VMEM((DPT,), jnp.float32),
        pltpu.VMEM((DPT,), jnp.float32),
        pltpu.VMEM_SHARED((NP,), jnp.float32),
        pltpu.VMEM_SHARED((NP,), jnp.float32),
    ],
)
def _prep(src_h, dst_h, ew_h, wn_h,
          src_a, ew_a, src_b, ew_b, src_c, dst_c, ew_c,
          dis_v, wn_v, deg_s, dis_sv, deg_sh, dis_sh):
    c = lax.axis_index("c")
    s = lax.axis_index("s")
    g = c * NS + s

    zeros16 = jnp.zeros((LANES,), jnp.float32)

    def z_body(i, _):
        deg_s[pl.ds(i * LANES, LANES)] = zeros16
        return 0
    lax.fori_loop(0, DPT // LANES, z_body, 0)
    pltpu.sync_copy(deg_s, deg_sh.at[pl.ds(s * DPT, DPT)])
    plsc.subcore_barrier()

    pltpu.sync_copy(src_h.at[s], src_a)
    pltpu.sync_copy(ew_h.at[s], ew_a)
    pltpu.sync_copy(src_h.at[s + NS], src_b)
    pltpu.sync_copy(ew_h.at[s + NS], ew_b)

    def deg_body(i, _):
        pltpu.sync_copy(ew_a.at[i], deg_sh.at[src_a.at[i]], add=True)
        pltpu.sync_copy(ew_b.at[i], deg_sh.at[src_b.at[i]], add=True)
        return 0
    lax.fori_loop(0, CPW, deg_body, 0)
    plsc.subcore_barrier()

    pltpu.sync_copy(deg_sh.at[pl.ds(s * DPT, DPT)], deg_s)

    def dis_body(i, _):
        d16 = deg_s[pl.ds(i * LANES, LANES)]
        r = _fast_rsqrt(d16)
        dis_sv[pl.ds(i * LANES, LANES)] = jnp.where(d16 > 0.0, r, 0.0)
        return 0
    lax.fori_loop(0, DPT // LANES, dis_body, 0)
    pltpu.sync_copy(dis_sv, dis_sh.at[pl.ds(s * DPT, DPT)])
    plsc.subcore_barrier()

    pltpu.sync_copy(dis_sh, dis_v)
    pltpu.sync_copy(src_h.at[g], src_c)
    pltpu.sync_copy(dst_h.at[g], dst_c)
    pltpu.sync_copy(ew_h.at[g], ew_c)

    def wn_body(i, _):
        def grp(k, _):
            sl = pl.ds(k * LANES, LANES)
            s16 = src_c[i, sl]
            d16 = dst_c[i, sl]
            w16 = ew_c[i, sl]
            a = plsc.load_gather(dis_v, [s16])
            b = plsc.load_gather(dis_v, [d16])
            wn_v[i, sl] = -(a * w16 * b)
            return 0
        lax.fori_loop(0, CHUNK // LANES, grp, 0)
        return 0
    lax.fori_loop(0, CPW, wn_body, 0)
    pltpu.sync_copy(wn_v, wn_h.at[g])


@functools.partial(
    pl.kernel,
    out_type=jax.ShapeDtypeStruct((NC, NP, D), jnp.float32),
    mesh=_mesh,
    compiler_params=_sc_params,
    scratch_types=[
        pltpu.VMEM((CPW, CHUNK), jnp.int32),
        pltpu.VMEM((CPW, CHUNK), jnp.int32),
        pltpu.VMEM((CPW, CHUNK), jnp.float32),
        pltpu.VMEM((CHUNK, D), jnp.float32),
        pltpu.VMEM_SHARED((NP, D), jnp.float32),
        pltpu.SemaphoreType.DMA,
    ],
)
def _prop(t_h, src_h, dst_h, wn_h, out_h, src_v, dst_v, wn_v, rows0,
          acc_sh, sg0):
    c = lax.axis_index("c")
    s = lax.axis_index("s")
    g = s * NC + c

    zeros16 = jnp.zeros((LANES,), jnp.float32)

    def zrow(i, _):
        for j in range(D // LANES):
            rows0[i, pl.ds(j * LANES, LANES)] = zeros16
        return 0
    lax.fori_loop(0, CHUNK, zrow, 0)
    for b in range(RPT // CHUNK):
        pltpu.sync_copy(rows0, acc_sh.at[pl.ds(s * RPT + b * CHUNK, CHUNK)])
    plsc.subcore_barrier()

    pltpu.sync_copy(src_h.at[g], src_v)
    pltpu.sync_copy(dst_h.at[g], dst_v)
    pltpu.sync_copy(wn_h.at[g], wn_v)

    def chunk_body(i, _):
        pltpu.async_copy(t_h.at[src_v.at[i]], rows0, sg0).wait()

        def edge(e, _):
            wb = plsc.load_gather(wn_v, [_splat(i), _splat(e)])
            for j in range(D // LANES):
                sl = pl.ds(j * LANES, LANES)
                rows0[e, sl] = rows0[e, sl] * wb
            return 0
        lax.fori_loop(0, CHUNK, edge, 0)
        pltpu.sync_copy(rows0, acc_sh.at[dst_v.at[i]], add=True)
        return 0
    lax.fori_loop(0, CPW, chunk_body, 0)
    plsc.subcore_barrier()

    for b in range(RPT // CHUNK):
        r0 = s * RPT + b * CHUNK
        pltpu.sync_copy(acc_sh.at[pl.ds(r0, CHUNK)],
                        out_h.at[c, pl.ds(r0, CHUNK), :])


_BLK = 512
_NBLK = NP // _BLK


def _combine1_body(p0_ref, p1_ref, o_ref):
    o_ref[...] = p0_ref[...] + p1_ref[...]


def _combine2_body(p0_ref, p1_ref, prev_ref, o_ref):
    o_ref[...] = 2.0 * (p0_ref[...] + p1_ref[...]) - prev_ref[...]


def _combine1(p0, p1):
    return pl.pallas_call(
        _combine1_body,
        out_shape=jax.ShapeDtypeStruct((NP, D), jnp.float32),
        grid=(_NBLK,),
        in_specs=[pl.BlockSpec((_BLK, D), lambda i: (i, 0))] * 2,
        out_specs=pl.BlockSpec((_BLK, D), lambda i: (i, 0)),
    )(p0, p1)


def _combine2(p0, p1, prev):
    return pl.pallas_call(
        _combine2_body,
        out_shape=jax.ShapeDtypeStruct((NP, D), jnp.float32),
        grid=(_NBLK,),
        in_specs=[pl.BlockSpec((_BLK, D), lambda i: (i, 0))] * 3,
        out_specs=pl.BlockSpec((_BLK, D), lambda i: (i, 0)),
    )(p0, p1, prev)


def _matmul5_body(relu, t0, t1, t2, t3, t4, w_ref, b_ref, o_ref):
    acc = jnp.dot(t0[...], w_ref[0], preferred_element_type=jnp.float32)
    for k, t in enumerate((t1, t2, t3, t4)):
        acc += jnp.dot(t[...], w_ref[k + 1],
                       preferred_element_type=jnp.float32)
    acc = acc + b_ref[...]
    if relu:
        acc = jnp.maximum(acc, 0.0)
    o_ref[...] = acc


def _matmul5(txs, W, b2d, relu):
    return pl.pallas_call(
        functools.partial(_matmul5_body, relu),
        out_shape=jax.ShapeDtypeStruct((NP, D), jnp.float32),
        grid=(_NBLK,),
        in_specs=[pl.BlockSpec((_BLK, D), lambda i: (i, 0))] * 5
        + [pl.BlockSpec((K, D, D), lambda i: (0, 0, 0)),
           pl.BlockSpec((1, D), lambda i: (0, 0))],
        out_specs=pl.BlockSpec((_BLK, D), lambda i: (i, 0)),
    )(*txs, W, b2d)


def _pool_head_body(h_ref, b3_ref, w1_ref, b1_ref, w2_ref, b2_ref, o_ref,
                    pooled, cnt):
    i = pl.program_id(0)

    @pl.when(i == 0)
    def _():
        pooled[...] = jnp.zeros_like(pooled)
        cnt[...] = jnp.zeros_like(cnt)

    bvec = b3_ref[0, 0, :]
    classes = lax.broadcasted_iota(jnp.int32, (_BLK, G), 1)
    onehot = (bvec[:, None] == classes).astype(jnp.float32)
    pooled[...] += jnp.dot(onehot.T, h_ref[...],
                           preferred_element_type=jnp.float32)
    cnt[...] += jnp.broadcast_to(
        jnp.sum(onehot, axis=0)[:, None], (G, D))

    @pl.when(i == _NBLK - 1)
    def _():
        p = pooled[...] / jnp.maximum(cnt[...], 1.0)
        p = jnp.maximum(p, 0.0)
        z = jnp.dot(p, w1_ref[...], preferred_element_type=jnp.float32)
        z = z + b1_ref[...]
        z = jnp.dot(z, w2_ref[...], preferred_element_type=jnp.float32)
        z = z + b2_ref[...]
        o_ref[...] = 1.0 / (1.0 + jnp.exp(-z))


def _pool_head(h, batch3, lin1_W, b1_2d, lin2_p, b2_2d):
    return pl.pallas_call(
        _pool_head_body,
        out_shape=jax.ShapeDtypeStruct((G, D), jnp.float32),
        grid=(_NBLK,),
        in_specs=[
            pl.BlockSpec((_BLK, D), lambda i: (i, 0)),
            pl.BlockSpec((1, 1, _BLK), lambda i: (i, 0, 0)),
            pl.BlockSpec((D, D), lambda i: (0, 0)),
            pl.BlockSpec((1, D), lambda i: (0, 0)),
            pl.BlockSpec((D, D), lambda i: (0, 0)),
            pl.BlockSpec((1, D), lambda i: (0, 0)),
        ],
        out_specs=pl.BlockSpec((G, D), lambda i: (0, 0)),
        scratch_shapes=[
            pltpu.VMEM((G, D), jnp.float32),
            pltpu.VMEM((G, D), jnp.float32),
        ],
    )(h, batch3, lin1_W, b1_2d, lin2_p, b2_2d)


def _cheb_layer(h, src3, dst3, wn3, W, b2d, relu):
    tx0 = h
    p = _prop(tx0, src3, dst3, wn3)
    tx1 = _combine1(p[0], p[1])
    p = _prop(tx1, src3, dst3, wn3)
    tx2 = _combine2(p[0], p[1], tx0)
    p = _prop(tx2, src3, dst3, wn3)
    tx3 = _combine2(p[0], p[1], tx1)
    p = _prop(tx3, src3, dst3, wn3)
    tx4 = _combine2(p[0], p[1], tx2)
    return _matmul5((tx0, tx1, tx2, tx3, tx4), W, b2d, relu)


def kernel(x, edge_index, edge_weight, batch,
           conv1_W, conv1_b, conv2_W, conv2_b, conv3_W, conv3_b,
           lin1_W, lin1_b, lin2_W, lin2_b):
    src = edge_index[0]
    dst = edge_index[1]
    pad_e = EP - E
    src3 = jnp.pad(src, (0, pad_e)).reshape(NW, CPW, CHUNK)
    dst3 = jnp.pad(dst, (0, pad_e)).reshape(NW, CPW, CHUNK)
    ew3 = jnp.pad(edge_weight, (0, pad_e)).reshape(NW, CPW, CHUNK)
    x_p = jnp.pad(x, ((0, NP - N), (0, 0)))
    batch3 = jnp.pad(batch, (0, NP - N), constant_values=G).reshape(
        _NBLK, 1, _BLK)
    b1c = conv1_b.reshape(1, D)
    b2c = conv2_b.reshape(1, D)
    b3c = conv3_b.reshape(1, D)
    l1b = lin1_b.reshape(1, D)
    l2w = jnp.pad(lin2_W, ((0, 0), (0, D - 1)))
    l2b = jnp.pad(lin2_b, (0, D - 1)).reshape(1, D)

    wn3 = _prep(src3, dst3, ew3)

    h = _cheb_layer(x_p, src3, dst3, wn3, conv1_W, b1c, relu=True)
    h = _cheb_layer(h, src3, dst3, wn3, conv2_W, b2c, relu=False)
    h = _cheb_layer(h, src3, dst3, wn3, conv3_W, b3c, relu=True)

    out = _pool_head(h, batch3, lin1_W, l1b, l2w, l2b)
    return out[:, :1]
